# Initial kernel scaffold; baseline (speedup 1.0000x reference)
import functools

import jax
import jax.numpy as jnp
from jax import lax
from jax.experimental import pallas as pl
from jax.experimental.pallas import tpu as pltpu
from jax.experimental.pallas import tpu_sc as plsc

NN = 10000
EE = 320000
DIN = 128
DH = 128
DOUT = 64

NPAD = 10240
NW = 32
EPW = 10240
EPAD = NW * EPW
CH = 128
NCH = EPW // CH
RPT = NPAD // 16

_MESH = dict(core_axis_name="c", subcore_axis_name="s")


def _deg_body(srcp, dstp, out, idxv, onesv, buf, dsrc_sh, ddst_sh):
    c = lax.axis_index("c")
    s = lax.axis_index("s")
    wid = s * 2 + c

    def zbody(i, _):
        buf[pl.ds(i * 16, 16)] = jnp.zeros((16,), jnp.float32)
        return 0

    lax.fori_loop(0, RPT // 16, zbody, 0)
    for i in range(CH // 16):
        onesv[pl.ds(i * 16, 16)] = jnp.ones((16,), jnp.float32)
    pltpu.sync_copy(buf, dsrc_sh.at[pl.ds(s * RPT, RPT)])
    pltpu.sync_copy(buf, ddst_sh.at[pl.ds(s * RPT, RPT)])
    plsc.subcore_barrier()

    base = wid * EPW

    def ebody(j, _):
        off = base + j * CH
        pltpu.sync_copy(srcp.at[pl.ds(off, CH)], idxv)
        pltpu.sync_copy(onesv, dsrc_sh.at[idxv], add=True)
        pltpu.sync_copy(dstp.at[pl.ds(off, CH)], idxv)
        pltpu.sync_copy(onesv, ddst_sh.at[idxv], add=True)
        return 0

    lax.fori_loop(0, NCH, ebody, 0)
    plsc.subcore_barrier()

    pltpu.sync_copy(dsrc_sh.at[pl.ds(s * RPT, RPT)], buf)
    pltpu.sync_copy(buf, out.at[c, 0, pl.ds(s * RPT, RPT)])
    pltpu.sync_copy(ddst_sh.at[pl.ds(s * RPT, RPT)], buf)
    pltpu.sync_copy(buf, out.at[c, 1, pl.ds(s * RPT, RPT)])


_deg_call = pl.kernel(
    _deg_body,
    out_type=jax.ShapeDtypeStruct((2, 2, NPAD), jnp.float32),
    mesh=plsc.VectorSubcoreMesh(**_MESH),
    scratch_types=[
        pltpu.VMEM((CH,), jnp.int32),
        pltpu.VMEM((CH,), jnp.float32),
        pltpu.VMEM((RPT,), jnp.float32),
        pltpu.VMEM_SHARED((NPAD,), jnp.float32),
        pltpu.VMEM_SHARED((NPAD,), jnp.float32),
    ],
)


def _make_prop(D):
    def _prop_body(p_hbm, srcp, dstp, out, isa, isb, ida, idb, rowsa, rowsb,
                   agg, sema, semb):
        c = lax.axis_index("c")
        s = lax.axis_index("s")
        wid = s * 2 + c

        def zrow(i, _):
            for k in range(D // 16):
                rowsa[i, pl.ds(k * 16, 16)] = jnp.zeros((16,), jnp.float32)
            return 0

        lax.fori_loop(0, CH, zrow, 0)
        for r in range(RPT // CH):
            pltpu.sync_copy(rowsa, agg.at[pl.ds(s * RPT + r * CH, CH)])
        plsc.subcore_barrier()

        base = wid * EPW
        pltpu.sync_copy(srcp.at[pl.ds(base, CH)], isa)
        pltpu.async_copy(p_hbm.at[isa], rowsa, sema)

        def body(t, _):
            j0 = 2 * t
            pltpu.sync_copy(srcp.at[pl.ds(base + (j0 + 1) * CH, CH)], isb)
            pltpu.async_copy(p_hbm.at[isb], rowsb, semb)
            pltpu.sync_copy(dstp.at[pl.ds(base + j0 * CH, CH)], ida)
            pltpu.make_async_copy(p_hbm.at[isa], rowsa, sema).wait()
            pltpu.sync_copy(rowsa, agg.at[ida], add=True)

            @pl.when(t + 1 < NCH // 2)
            def _():
                pltpu.sync_copy(srcp.at[pl.ds(base + (j0 + 2) * CH, CH)], isa)
                pltpu.async_copy(p_hbm.at[isa], rowsa, sema)

            pltpu.sync_copy(dstp.at[pl.ds(base + (j0 + 1) * CH, CH)], idb)
            pltpu.make_async_copy(p_hbm.at[isb], rowsb, semb).wait()
            pltpu.sync_copy(rowsb, agg.at[idb], add=True)
            return 0

        lax.fori_loop(0, NCH // 2, body, 0)
        plsc.subcore_barrier()

        for r in range(RPT // CH):
            off = s * RPT + r * CH
            pltpu.sync_copy(agg.at[pl.ds(off, CH)], rowsa)
            pltpu.sync_copy(rowsa, out.at[c, pl.ds(off, CH)])

    return pl.kernel(
        _prop_body,
        out_type=jax.ShapeDtypeStruct((2, NPAD, D), jnp.float32),
        mesh=plsc.VectorSubcoreMesh(**_MESH),
        scratch_types=[
            pltpu.VMEM((CH,), jnp.int32),
            pltpu.VMEM((CH,), jnp.int32),
            pltpu.VMEM((CH,), jnp.int32),
            pltpu.VMEM((CH,), jnp.int32),
            pltpu.VMEM((CH, D), jnp.float32),
            pltpu.VMEM((CH, D), jnp.float32),
            pltpu.VMEM_SHARED((NPAD, D), jnp.float32),
            pltpu.SemaphoreType.DMA,
            pltpu.SemaphoreType.DMA,
        ],
    )


_prop_h = _make_prop(DH)
_prop_o = _make_prop(DOUT)


BR = 512


def _tc1_body(deg_ref, x_ref, w1_ref, p1_ref, nrm_ref):
    d = deg_ref[...]
    dsrc = d[0, 0] + d[1, 0]
    ddst = d[0, 1] + d[1, 1]
    ns = jnp.where(dsrc > 0, lax.rsqrt(jnp.maximum(dsrc, 1.0)), 0.0)
    nd = jnp.where(ddst > 0, lax.rsqrt(jnp.maximum(ddst, 1.0)), 0.0)
    nrm_ref[0] = ns
    nrm_ref[1] = nd
    xw = jnp.dot(x_ref[...], w1_ref[...], preferred_element_type=jnp.float32)
    p1_ref[...] = xw * ns


def _tc2_body(agg_ref, nrm_ref, b1_ref, w2_ref, p2_ref):
    a = agg_ref[0] + agg_ref[1]
    h = jnp.maximum(a * nrm_ref[1] + b1_ref[...], 0.0)
    hw = jnp.dot(h, w2_ref[...], preferred_element_type=jnp.float32)
    p2_ref[...] = hw * nrm_ref[0]


def _tc3_body(agg_ref, nrm_ref, b2_ref, o_ref):
    a = agg_ref[0] + agg_ref[1]
    o_ref[...] = a * nrm_ref[1] + b2_ref[...]


_GRID = (NPAD // BR,)

_tc1 = pl.pallas_call(
    _tc1_body,
    grid=_GRID,
    in_specs=[
        pl.BlockSpec((2, 2, BR, 1), lambda i: (0, 0, i, 0)),
        pl.BlockSpec((BR, DIN), lambda i: (i, 0)),
        pl.BlockSpec((DIN, DH), lambda i: (0, 0)),
    ],
    out_specs=[
        pl.BlockSpec((BR, DH), lambda i: (i, 0)),
        pl.BlockSpec((2, BR, 1), lambda i: (0, i, 0)),
    ],
    out_shape=[
        jax.ShapeDtypeStruct((NPAD, DH), jnp.float32),
        jax.ShapeDtypeStruct((2, NPAD, 1), jnp.float32),
    ],
)

_tc2 = pl.pallas_call(
    _tc2_body,
    grid=_GRID,
    in_specs=[
        pl.BlockSpec((2, BR, DH), lambda i: (0, i, 0)),
        pl.BlockSpec((2, BR, 1), lambda i: (0, i, 0)),
        pl.BlockSpec((1, DH), lambda i: (0, 0)),
        pl.BlockSpec((DH, DOUT), lambda i: (0, 0)),
    ],
    out_specs=pl.BlockSpec((BR, DOUT), lambda i: (i, 0)),
    out_shape=jax.ShapeDtypeStruct((NPAD, DOUT), jnp.float32),
)

_tc3 = pl.pallas_call(
    _tc3_body,
    grid=_GRID,
    in_specs=[
        pl.BlockSpec((2, BR, DOUT), lambda i: (0, i, 0)),
        pl.BlockSpec((2, BR, 1), lambda i: (0, i, 0)),
        pl.BlockSpec((1, DOUT), lambda i: (0, 0)),
    ],
    out_specs=pl.BlockSpec((BR, DOUT), lambda i: (i, 0)),
    out_shape=jax.ShapeDtypeStruct((NPAD, DOUT), jnp.float32),
)


def kernel(features, edge_index, W1, b1, W2, b2):
    pad = jnp.full((EPAD - EE,), NN, jnp.int32)
    srcp = jnp.concatenate([edge_index[0], pad])
    dstp = jnp.concatenate([edge_index[1], pad])
    x_pad = jnp.pad(features, ((0, NPAD - NN), (0, 0)))

    degp = _deg_call(srcp, dstp)
    degcol = degp.reshape(2, 2, NPAD, 1)
    p1, nrm = _tc1(degcol, x_pad, W1)
    agg1 = _prop_h(p1, srcp, dstp)
    p2 = _tc2(agg1, nrm, b1.reshape(1, DH), W2)
    agg2 = _prop_o(p2, srcp, dstp)
    outp = _tc3(agg2, nrm, b2.reshape(1, DOUT))
    return outp[:NN]


# speedup vs baseline: 4.2684x; 4.2684x over previous
import functools

import jax
import jax.numpy as jnp
from jax import lax
from jax.experimental import pallas as pl
from jax.experimental.pallas import tpu as pltpu
from jax.experimental.pallas import tpu_sc as plsc

NN = 10000
EE = 320000
DIN = 128
DH = 128
DOUT = 64

NPAD = 10240
NW = 32
EPW = 10240
EPAD = NW * EPW
CH = 128
NCH = EPW // CH
RPT = NPAD // 16

_MESH = dict(core_axis_name="c", subcore_axis_name="s")


def _deg_body(srcp, dstp, out, idxv, onesv, buf, dsrc_sh, ddst_sh):
    c = lax.axis_index("c")
    s = lax.axis_index("s")
    wid = s * 2 + c

    def zbody(i, _):
        buf[pl.ds(i * 16, 16)] = jnp.zeros((16,), jnp.float32)
        return 0

    lax.fori_loop(0, RPT // 16, zbody, 0)
    for i in range(CH // 16):
        onesv[pl.ds(i * 16, 16)] = jnp.ones((16,), jnp.float32)
    pltpu.sync_copy(buf, dsrc_sh.at[pl.ds(s * RPT, RPT)])
    pltpu.sync_copy(buf, ddst_sh.at[pl.ds(s * RPT, RPT)])
    plsc.subcore_barrier()

    base = wid * EPW

    def ebody(j, _):
        off = base + j * CH
        pltpu.sync_copy(srcp.at[pl.ds(off, CH)], idxv)
        pltpu.sync_copy(onesv, dsrc_sh.at[idxv], add=True)
        pltpu.sync_copy(dstp.at[pl.ds(off, CH)], idxv)
        pltpu.sync_copy(onesv, ddst_sh.at[idxv], add=True)
        return 0

    lax.fori_loop(0, NCH, ebody, 0)
    plsc.subcore_barrier()

    pltpu.sync_copy(dsrc_sh.at[pl.ds(s * RPT, RPT)], buf)
    pltpu.sync_copy(buf, out.at[c, 0, pl.ds(s * RPT, RPT)])
    pltpu.sync_copy(ddst_sh.at[pl.ds(s * RPT, RPT)], buf)
    pltpu.sync_copy(buf, out.at[c, 1, pl.ds(s * RPT, RPT)])


_deg_call = pl.kernel(
    _deg_body,
    out_type=jax.ShapeDtypeStruct((2, 2, NPAD), jnp.float32),
    mesh=plsc.VectorSubcoreMesh(**_MESH),
    scratch_types=[
        pltpu.VMEM((CH,), jnp.int32),
        pltpu.VMEM((CH,), jnp.float32),
        pltpu.VMEM((RPT,), jnp.float32),
        pltpu.VMEM_SHARED((NPAD,), jnp.float32),
        pltpu.VMEM_SHARED((NPAD,), jnp.float32),
    ],
)


def _make_prop(D, tc_tiling=True):
    def _prop_body(p_hbm, srcp, dstp, out, isa, isb, ida, idb, rowsa, rowsb,
                   agg, sema, semb):
        c = lax.axis_index("c")
        s = lax.axis_index("s")
        wid = s * 2 + c

        def zrow(i, _):
            for k in range(D // 16):
                rowsa[i, pl.ds(k * 16, 16)] = jnp.zeros((16,), jnp.float32)
            return 0

        lax.fori_loop(0, CH, zrow, 0)
        for r in range(RPT // CH):
            pltpu.sync_copy(rowsa, agg.at[pl.ds(s * RPT + r * CH, CH)])
        plsc.subcore_barrier()

        base = wid * EPW
        pltpu.sync_copy(srcp.at[pl.ds(base, CH)], isa)
        pltpu.async_copy(p_hbm.at[isa], rowsa, sema)

        def body(t, _):
            j0 = 2 * t
            pltpu.sync_copy(srcp.at[pl.ds(base + (j0 + 1) * CH, CH)], isb)
            pltpu.async_copy(p_hbm.at[isb], rowsb, semb)
            pltpu.sync_copy(dstp.at[pl.ds(base + j0 * CH, CH)], ida)
            pltpu.make_async_copy(p_hbm.at[isa], rowsa, sema).wait()
            pltpu.sync_copy(rowsa, agg.at[ida], add=True)

            @pl.when(t + 1 < NCH // 2)
            def _():
                pltpu.sync_copy(srcp.at[pl.ds(base + (j0 + 2) * CH, CH)], isa)
                pltpu.async_copy(p_hbm.at[isa], rowsa, sema)

            pltpu.sync_copy(dstp.at[pl.ds(base + (j0 + 1) * CH, CH)], idb)
            pltpu.make_async_copy(p_hbm.at[isb], rowsb, semb).wait()
            pltpu.sync_copy(rowsb, agg.at[idb], add=True)
            return 0

        lax.fori_loop(0, NCH // 2, body, 0)
        plsc.subcore_barrier()

        for r in range(RPT // CH):
            off = s * RPT + r * CH
            pltpu.sync_copy(agg.at[pl.ds(off, CH)], rowsa)
            pltpu.sync_copy(rowsa, out.at[c, pl.ds(off, CH)])

    return pl.kernel(
        _prop_body,
        out_type=jax.ShapeDtypeStruct((2, NPAD, D), jnp.float32),
        mesh=plsc.VectorSubcoreMesh(**_MESH),
        compiler_params=pltpu.CompilerParams(use_tc_tiling_on_sc=tc_tiling),
        scratch_types=[
            pltpu.VMEM((CH,), jnp.int32),
            pltpu.VMEM((CH,), jnp.int32),
            pltpu.VMEM((CH,), jnp.int32),
            pltpu.VMEM((CH,), jnp.int32),
            pltpu.VMEM((CH, D), jnp.float32),
            pltpu.VMEM((CH, D), jnp.float32),
            pltpu.VMEM_SHARED((NPAD, D), jnp.float32),
            pltpu.SemaphoreType.DMA,
            pltpu.SemaphoreType.DMA,
        ],
    )


_prop_h = _make_prop(DH)
_prop_o = _make_prop(DOUT, tc_tiling=False)


BR = 512


def _tc1_body(deg_ref, x_ref, w1_ref, p1_ref, nrm_ref):
    d = deg_ref[...]
    dsrc = d[0, 0] + d[1, 0]
    ddst = d[0, 1] + d[1, 1]
    ns = jnp.where(dsrc > 0, lax.rsqrt(jnp.maximum(dsrc, 1.0)), 0.0)
    nd = jnp.where(ddst > 0, lax.rsqrt(jnp.maximum(ddst, 1.0)), 0.0)
    nrm_ref[0] = ns
    nrm_ref[1] = nd
    xw = jnp.dot(x_ref[...], w1_ref[...], preferred_element_type=jnp.float32)
    p1_ref[...] = xw * ns


def _tc2_body(agg_ref, nrm_ref, b1_ref, w2_ref, p2_ref):
    a = agg_ref[0] + agg_ref[1]
    h = jnp.maximum(a * nrm_ref[1] + b1_ref[...], 0.0)
    hw = jnp.dot(h, w2_ref[...], preferred_element_type=jnp.float32)
    p2_ref[...] = hw * nrm_ref[0]


def _tc3_body(agg_ref, nrm_ref, b2_ref, o_ref):
    a = agg_ref[0] + agg_ref[1]
    o_ref[...] = a * nrm_ref[1] + b2_ref[...]


_GRID = (NPAD // BR,)

_tc1 = pl.pallas_call(
    _tc1_body,
    grid=_GRID,
    in_specs=[
        pl.BlockSpec((2, 2, BR, 1), lambda i: (0, 0, i, 0)),
        pl.BlockSpec((BR, DIN), lambda i: (i, 0)),
        pl.BlockSpec((DIN, DH), lambda i: (0, 0)),
    ],
    out_specs=[
        pl.BlockSpec((BR, DH), lambda i: (i, 0)),
        pl.BlockSpec((2, BR, 1), lambda i: (0, i, 0)),
    ],
    out_shape=[
        jax.ShapeDtypeStruct((NPAD, DH), jnp.float32),
        jax.ShapeDtypeStruct((2, NPAD, 1), jnp.float32),
    ],
)

_tc2 = pl.pallas_call(
    _tc2_body,
    grid=_GRID,
    in_specs=[
        pl.BlockSpec((2, BR, DH), lambda i: (0, i, 0)),
        pl.BlockSpec((2, BR, 1), lambda i: (0, i, 0)),
        pl.BlockSpec((1, DH), lambda i: (0, 0)),
        pl.BlockSpec((DH, DOUT), lambda i: (0, 0)),
    ],
    out_specs=pl.BlockSpec((BR, DOUT), lambda i: (i, 0)),
    out_shape=jax.ShapeDtypeStruct((NPAD, DOUT), jnp.float32),
)

_tc3 = pl.pallas_call(
    _tc3_body,
    grid=_GRID,
    in_specs=[
        pl.BlockSpec((2, BR, DOUT), lambda i: (0, i, 0)),
        pl.BlockSpec((2, BR, 1), lambda i: (0, i, 0)),
        pl.BlockSpec((1, DOUT), lambda i: (0, 0)),
    ],
    out_specs=pl.BlockSpec((BR, DOUT), lambda i: (i, 0)),
    out_shape=jax.ShapeDtypeStruct((NPAD, DOUT), jnp.float32),
)


def kernel(features, edge_index, W1, b1, W2, b2):
    pad = jnp.full((EPAD - EE,), NN, jnp.int32)
    srcp = jnp.concatenate([edge_index[0], pad])
    dstp = jnp.concatenate([edge_index[1], pad])
    x_pad = jnp.pad(features, ((0, NPAD - NN), (0, 0)))

    degp = _deg_call(srcp, dstp)
    degcol = degp.reshape(2, 2, NPAD, 1)
    p1, nrm = _tc1(degcol, x_pad, W1)
    agg1 = _prop_h(p1, srcp, dstp)
    p2 = _tc2(agg1, nrm, b1.reshape(1, DH), W2)
    agg2 = _prop_o(p2, srcp, dstp)
    outp = _tc3(agg2, nrm, b2.reshape(1, DOUT))
    return outp[:NN]


# speedup vs baseline: 5.7128x; 1.3384x over previous
import jax
import jax.numpy as jnp
from jax import lax
from jax.experimental import pallas as pl
from jax.experimental.pallas import tpu as pltpu
from jax.experimental.pallas import tpu_sc as plsc

NN = 10000
EE = 320000
DIN = 128
DH = 128
DOUT = 64

NPAD = 10240
CH = 128
EPAD = 327680
NCHD = 80
NCHP = 160
RPT = NPAD // 16

_MESH = dict(core_axis_name="c", subcore_axis_name="s")


def _deg_body(srcp3, dstp3, out, sidx, didx, onesv, buf, dsrc_sh, ddst_sh,
              sems, semd):
    c = lax.axis_index("c")
    s = lax.axis_index("s")
    wid = s * 2 + c

    def zbody(i, _):
        buf[pl.ds(i * 16, 16)] = jnp.zeros((16,), jnp.float32)
        return 0

    lax.fori_loop(0, RPT // 16, zbody, 0)
    for i in range(CH // 16):
        onesv[pl.ds(i * 16, 16)] = jnp.ones((16,), jnp.float32)
    pltpu.sync_copy(srcp3.at[wid], sidx)
    pltpu.sync_copy(dstp3.at[wid], didx)
    pltpu.sync_copy(buf, dsrc_sh.at[pl.ds(s * RPT, RPT)])
    pltpu.sync_copy(buf, ddst_sh.at[pl.ds(s * RPT, RPT)])
    plsc.subcore_barrier()

    def ebody(t, _):
        pltpu.async_copy(onesv, dsrc_sh.at[sidx.at[t]], sems, add=True)
        pltpu.async_copy(onesv, ddst_sh.at[didx.at[t]], semd, add=True)

        @pl.when(t > 0)
        def _():
            pltpu.make_async_copy(onesv, dsrc_sh.at[sidx.at[t - 1]], sems).wait()
            pltpu.make_async_copy(onesv, ddst_sh.at[didx.at[t - 1]], semd).wait()

        return 0

    lax.fori_loop(0, NCHD, ebody, 0)
    pltpu.make_async_copy(onesv, dsrc_sh.at[sidx.at[NCHD - 1]], sems).wait()
    pltpu.make_async_copy(onesv, ddst_sh.at[didx.at[NCHD - 1]], semd).wait()
    plsc.subcore_barrier()

    pltpu.sync_copy(dsrc_sh.at[pl.ds(s * RPT, RPT)], buf)
    pltpu.sync_copy(buf, out.at[c, 0, pl.ds(s * RPT, RPT)])
    pltpu.sync_copy(ddst_sh.at[pl.ds(s * RPT, RPT)], buf)
    pltpu.sync_copy(buf, out.at[c, 1, pl.ds(s * RPT, RPT)])


_deg_call = pl.kernel(
    _deg_body,
    out_type=jax.ShapeDtypeStruct((2, 2, NPAD), jnp.float32),
    mesh=plsc.VectorSubcoreMesh(**_MESH),
    scratch_types=[
        pltpu.VMEM((NCHD, CH), jnp.int32),
        pltpu.VMEM((NCHD, CH), jnp.int32),
        pltpu.VMEM((CH,), jnp.float32),
        pltpu.VMEM((RPT,), jnp.float32),
        pltpu.VMEM_SHARED((NPAD,), jnp.float32),
        pltpu.VMEM_SHARED((NPAD,), jnp.float32),
        pltpu.SemaphoreType.DMA,
        pltpu.SemaphoreType.DMA,
    ],
)


NBUF = 4


def _make_prop(D2):
    def _prop_body(p_hbm, srcp4, dstp3, out, sidx, didx, r0, r1, r2, r3,
                   agg, g0, g1, g2, g3, c0, c1, c2, c3):
        c = lax.axis_index("c")
        s = lax.axis_index("s")
        rows = (r0, r1, r2, r3)
        gsem = (g0, g1, g2, g3)
        csem = (c0, c1, c2, c3)

        def zrow(i, _):
            for k in range(D2 // 16):
                r0[i, pl.ds(k * 16, 16)] = jnp.zeros((16,), jnp.float32)
            return 0

        lax.fori_loop(0, CH, zrow, 0)
        for r in range(RPT // CH):
            pltpu.sync_copy(r0, agg.at[pl.ds(s * RPT + r * CH, CH)])
        pltpu.sync_copy(srcp4.at[c, s], sidx)
        pltpu.sync_copy(dstp3.at[s], didx)
        plsc.subcore_barrier()

        for b in range(NBUF - 1):
            pltpu.async_copy(p_hbm.at[sidx.at[b]], rows[b], gsem[b])

        def body(t, _):
            j0 = NBUF * t
            for b in range(NBUF):
                j = j0 + b
                bn = (b + NBUF - 1) % NBUF

                @pl.when(j + NBUF - 1 < NCHP)
                def _(j=j, b=b, bn=bn):
                    @pl.when(j > 0)
                    def _():
                        pltpu.make_async_copy(
                            rows[bn], agg.at[didx.at[j - 1]], csem[bn]).wait()

                    pltpu.async_copy(
                        p_hbm.at[sidx.at[j + NBUF - 1]], rows[bn], gsem[bn])

                pltpu.make_async_copy(
                    p_hbm.at[sidx.at[j]], rows[b], gsem[b]).wait()
                pltpu.async_copy(rows[b], agg.at[didx.at[j]], csem[b],
                                 add=True)
            return 0

        lax.fori_loop(0, NCHP // NBUF, body, 0)
        for b in range(NBUF):
            j = NCHP - NBUF + b
            pltpu.make_async_copy(rows[b], agg.at[didx.at[j]], csem[b]).wait()
        plsc.subcore_barrier()

        for r in range(RPT // CH):
            off = s * RPT + r * CH
            pltpu.sync_copy(agg.at[pl.ds(off, CH)], r0)
            pltpu.sync_copy(r0, out.at[c, pl.ds(off, CH)])

    return pl.kernel(
        _prop_body,
        out_type=jax.ShapeDtypeStruct((2, NPAD, D2), jnp.float32),
        mesh=plsc.VectorSubcoreMesh(**_MESH),
        compiler_params=pltpu.CompilerParams(use_tc_tiling_on_sc=False),
        scratch_types=(
            [
                pltpu.VMEM((NCHP, CH), jnp.int32),
                pltpu.VMEM((NCHP, CH), jnp.int32),
            ]
            + [pltpu.VMEM((CH, D2), jnp.float32)] * NBUF
            + [pltpu.VMEM_SHARED((NPAD, D2), jnp.float32)]
            + [pltpu.SemaphoreType.DMA] * (2 * NBUF)
        ),
    )


_prop_h = _make_prop(DH // 2)
_prop_o = _make_prop(DOUT // 2)


BR = 512
DH2 = DH // 2
DO2 = DOUT // 2


def _tc1_body(deg_ref, x_ref, w1_ref, p1_ref, nrm_ref):
    d = deg_ref[...]
    dsrc = d[0, 0] + d[1, 0]
    ddst = d[0, 1] + d[1, 1]
    ns = jnp.where(dsrc > 0, lax.rsqrt(jnp.maximum(dsrc, 1.0)), 0.0)
    nd = jnp.where(ddst > 0, lax.rsqrt(jnp.maximum(ddst, 1.0)), 0.0)
    nrm_ref[0] = ns
    nrm_ref[1] = nd
    xw = jnp.dot(x_ref[...], w1_ref[...], preferred_element_type=jnp.float32)
    p1 = xw * ns
    p1_ref[0] = p1[:, :DH2]
    p1_ref[1] = p1[:, DH2:]


def _tc2_body(agg_ref, nrm_ref, b1_ref, w2_ref, p2_ref):
    a = jnp.concatenate([agg_ref[0], agg_ref[1]], axis=-1)
    h = jnp.maximum(a * nrm_ref[1] + b1_ref[...], 0.0)
    hw = jnp.dot(h, w2_ref[...], preferred_element_type=jnp.float32)
    p2 = hw * nrm_ref[0]
    p2_ref[0] = p2[:, :DO2]
    p2_ref[1] = p2[:, DO2:]


def _tc3_body(agg_ref, nrm_ref, b2_ref, o_ref):
    a = jnp.concatenate([agg_ref[0], agg_ref[1]], axis=-1)
    o_ref[...] = a * nrm_ref[1] + b2_ref[...]


_GRID = (NPAD // BR,)

_tc1 = pl.pallas_call(
    _tc1_body,
    grid=_GRID,
    in_specs=[
        pl.BlockSpec((2, 2, BR, 1), lambda i: (0, 0, i, 0)),
        pl.BlockSpec((BR, DIN), lambda i: (i, 0)),
        pl.BlockSpec((DIN, DH), lambda i: (0, 0)),
    ],
    out_specs=[
        pl.BlockSpec((2, BR, DH2), lambda i: (0, i, 0)),
        pl.BlockSpec((2, BR, 1), lambda i: (0, i, 0)),
    ],
    out_shape=[
        jax.ShapeDtypeStruct((2, NPAD, DH2), jnp.float32),
        jax.ShapeDtypeStruct((2, NPAD, 1), jnp.float32),
    ],
)

_tc2 = pl.pallas_call(
    _tc2_body,
    grid=_GRID,
    in_specs=[
        pl.BlockSpec((2, BR, DH2), lambda i: (0, i, 0)),
        pl.BlockSpec((2, BR, 1), lambda i: (0, i, 0)),
        pl.BlockSpec((1, DH), lambda i: (0, 0)),
        pl.BlockSpec((DH, DOUT), lambda i: (0, 0)),
    ],
    out_specs=pl.BlockSpec((2, BR, DO2), lambda i: (0, i, 0)),
    out_shape=jax.ShapeDtypeStruct((2, NPAD, DO2), jnp.float32),
)

_tc3 = pl.pallas_call(
    _tc3_body,
    grid=_GRID,
    in_specs=[
        pl.BlockSpec((2, BR, DO2), lambda i: (0, i, 0)),
        pl.BlockSpec((2, BR, 1), lambda i: (0, i, 0)),
        pl.BlockSpec((1, DOUT), lambda i: (0, 0)),
    ],
    out_specs=pl.BlockSpec((BR, DOUT), lambda i: (i, 0)),
    out_shape=jax.ShapeDtypeStruct((NPAD, DOUT), jnp.float32),
)


def kernel(features, edge_index, W1, b1, W2, b2):
    pad = jnp.full((EPAD - EE,), NN, jnp.int32)
    srcp = jnp.concatenate([edge_index[0], pad])
    dstp = jnp.concatenate([edge_index[1], pad])
    srcp_deg = srcp.reshape(32, NCHD, CH)
    dstp_deg = dstp.reshape(32, NCHD, CH)
    srcp_off = jnp.stack([srcp, srcp + NPAD]).reshape(2, 16, NCHP, CH)
    dstp_t = dstp.reshape(16, NCHP, CH)
    x_pad = jnp.pad(features, ((0, NPAD - NN), (0, 0)))

    degp = _deg_call(srcp_deg, dstp_deg)
    degcol = degp.reshape(2, 2, NPAD, 1)
    p1, nrm = _tc1(degcol, x_pad, W1)
    agg1 = _prop_h(p1.reshape(2 * NPAD, DH2), srcp_off, dstp_t)
    p2 = _tc2(agg1, nrm, b1.reshape(1, DH), W2)
    agg2 = _prop_o(p2.reshape(2 * NPAD, DO2), srcp_off, dstp_t)
    outp = _tc3(agg2, nrm, b2.reshape(1, DOUT))
    return outp[:NN]


# speedup vs baseline: 9.3282x; 1.6329x over previous
import jax
import jax.numpy as jnp
from jax import lax
from jax.experimental import pallas as pl
from jax.experimental.pallas import tpu as pltpu
from jax.experimental.pallas import tpu_sc as plsc

NN = 10000
EE = 320000
DIN = 128
DH = 128
DOUT = 64

NPAD = 10240
CH = 128
EPAD = 327680
NCHD = 80
NCHP = 160
RPT = NPAD // 16

_MESH = dict(core_axis_name="c", subcore_axis_name="s")


def _deg_body(srcp3, dstp3, out, sidx, didx, onesv, buf, dsrc_sh, ddst_sh,
              sems, semd):
    c = lax.axis_index("c")
    s = lax.axis_index("s")
    wid = s * 2 + c

    def zbody(i, _):
        buf[pl.ds(i * 16, 16)] = jnp.zeros((16,), jnp.float32)
        return 0

    lax.fori_loop(0, RPT // 16, zbody, 0)
    for i in range(CH // 16):
        onesv[pl.ds(i * 16, 16)] = jnp.ones((16,), jnp.float32)
    pltpu.sync_copy(srcp3.at[wid], sidx)
    pltpu.sync_copy(dstp3.at[wid], didx)
    pltpu.sync_copy(buf, dsrc_sh.at[pl.ds(s * RPT, RPT)])
    pltpu.sync_copy(buf, ddst_sh.at[pl.ds(s * RPT, RPT)])
    plsc.subcore_barrier()

    def ebody(t, _):
        pltpu.async_copy(onesv, dsrc_sh.at[sidx.at[t]], sems, add=True)
        pltpu.async_copy(onesv, ddst_sh.at[didx.at[t]], semd, add=True)

        @pl.when(t > 0)
        def _():
            pltpu.make_async_copy(onesv, dsrc_sh.at[sidx.at[t - 1]], sems).wait()
            pltpu.make_async_copy(onesv, ddst_sh.at[didx.at[t - 1]], semd).wait()

        return 0

    lax.fori_loop(0, NCHD, ebody, 0)
    pltpu.make_async_copy(onesv, dsrc_sh.at[sidx.at[NCHD - 1]], sems).wait()
    pltpu.make_async_copy(onesv, ddst_sh.at[didx.at[NCHD - 1]], semd).wait()
    plsc.subcore_barrier()

    pltpu.sync_copy(dsrc_sh.at[pl.ds(s * RPT, RPT)], buf)
    pltpu.sync_copy(buf, out.at[c, 0, pl.ds(s * RPT, RPT)])
    pltpu.sync_copy(ddst_sh.at[pl.ds(s * RPT, RPT)], buf)
    pltpu.sync_copy(buf, out.at[c, 1, pl.ds(s * RPT, RPT)])


_deg_call = pl.kernel(
    _deg_body,
    out_type=jax.ShapeDtypeStruct((2, 2, NPAD), jnp.float32),
    mesh=plsc.VectorSubcoreMesh(**_MESH),
    scratch_types=[
        pltpu.VMEM((NCHD, CH), jnp.int32),
        pltpu.VMEM((NCHD, CH), jnp.int32),
        pltpu.VMEM((CH,), jnp.float32),
        pltpu.VMEM((RPT,), jnp.float32),
        pltpu.VMEM_SHARED((NPAD,), jnp.float32),
        pltpu.VMEM_SHARED((NPAD,), jnp.float32),
        pltpu.SemaphoreType.DMA,
        pltpu.SemaphoreType.DMA,
    ],
)


def _make_prop(D2, nbuf):
    IBK = 16
    NBLK = NCHP // IBK

    def _prop_body(p_hbm, srcp3, dstp3, out, sidx, di0, di1, r0, r1,
                   tbl, agg, g0, g1, c0, c1, i0, i1):
        rows = (r0, r1)
        dbuf = (di0, di1)
        gsem = (g0, g1)
        csem = (c0, c1)
        isem = (i0, i1)
        c = lax.axis_index("c")
        s = lax.axis_index("s")

        def zrow(i, _):
            for k in range(D2 // 16):
                r0[i, pl.ds(k * 16, 16)] = jnp.zeros((16,), jnp.float32)
            return 0

        lax.fori_loop(0, CH, zrow, 0)
        for r in range(RPT // CH):
            pltpu.sync_copy(r0, agg.at[pl.ds(s * RPT + r * CH, CH)])
        for r in range(RPT // CH):
            off = s * RPT + r * CH
            pltpu.sync_copy(p_hbm.at[c, pl.ds(off, CH)], r0)
            pltpu.sync_copy(r0, tbl.at[pl.ds(off, CH)])
        pltpu.sync_copy(srcp3.at[s], sidx)
        pltpu.sync_copy(dstp3.at[s, pl.ds(0, IBK)], di0)
        plsc.subcore_barrier()

        pltpu.async_copy(tbl.at[sidx.at[0]], r0, g0)

        def _dblk(blk):
            return dstp3.at[s, pl.ds(blk * IBK, IBK)]

        def pair_body(u, _):
            for par in range(2):
                blk = 2 * u + par
                j0 = blk * IBK
                dref = dbuf[par]
                oref = dbuf[1 - par]
                if par == 0:
                    @pl.when(u > 0)
                    def _(oref=oref):
                        pltpu.make_async_copy(
                            rows[1], agg.at[oref.at[IBK - 1]], csem[1]).wait()
                else:
                    pltpu.make_async_copy(
                        rows[1], agg.at[oref.at[IBK - 1]], csem[1]).wait()

                @pl.when(blk + 1 < NBLK)
                def _(blk=blk, oref=oref, par=par):
                    pltpu.async_copy(_dblk(blk + 1), oref, isem[1 - par])

                if par == 0:
                    @pl.when(u > 0)
                    def _(dref=dref, par=par, blk=blk):
                        pltpu.make_async_copy(_dblk(blk), dref,
                                              isem[par]).wait()
                else:
                    pltpu.make_async_copy(_dblk(blk), dref, isem[par]).wait()

                for q in range(IBK):
                    j = j0 + q
                    b = q % 2
                    pltpu.make_async_copy(
                        tbl.at[sidx.at[j]], rows[b], gsem[b]).wait()
                    pltpu.async_copy(rows[b], agg.at[dref.at[q]], csem[b],
                                     add=True)
                    if q >= 1:
                        pltpu.make_async_copy(
                            rows[1 - b], agg.at[dref.at[q - 1]],
                            csem[1 - b]).wait()

                    @pl.when(j + 1 < NCHP)
                    def _(j=j, b=b):
                        pltpu.async_copy(
                            tbl.at[sidx.at[j + 1]], rows[1 - b], gsem[1 - b])
            return 0

        lax.fori_loop(0, NBLK // 2, pair_body, 0)
        pltpu.make_async_copy(
            rows[1], agg.at[di1.at[IBK - 1]], csem[1]).wait()
        plsc.subcore_barrier()

        for r in range(RPT // CH):
            off = s * RPT + r * CH
            pltpu.sync_copy(agg.at[pl.ds(off, CH)], r0)
            pltpu.sync_copy(r0, out.at[c, pl.ds(off, CH)])

    return pl.kernel(
        _prop_body,
        out_type=jax.ShapeDtypeStruct((2, NPAD, D2), jnp.float32),
        mesh=plsc.VectorSubcoreMesh(**_MESH),
        compiler_params=pltpu.CompilerParams(use_tc_tiling_on_sc=False),
        scratch_types=(
            [pltpu.VMEM((NCHP, CH), jnp.int32)]
            + [pltpu.VMEM((IBK, CH), jnp.int32)] * 2
            + [pltpu.VMEM((CH, D2), jnp.float32)] * 2
            + [pltpu.VMEM_SHARED((NPAD, D2), jnp.float32)] * 2
            + [pltpu.SemaphoreType.DMA] * 6
        ),
    )


_prop_h = _make_prop(DH // 2, 2)
_prop_o = _make_prop(DOUT // 2, 2)


BR = 512
DH2 = DH // 2
DO2 = DOUT // 2


def _tc1_body(deg_ref, x_ref, w1_ref, p1_ref, nrm_ref):
    d = deg_ref[...]
    dsrc = d[0, 0] + d[1, 0]
    ddst = d[0, 1] + d[1, 1]
    ns = jnp.where(dsrc > 0, lax.rsqrt(jnp.maximum(dsrc, 1.0)), 0.0)
    nd = jnp.where(ddst > 0, lax.rsqrt(jnp.maximum(ddst, 1.0)), 0.0)
    nrm_ref[0] = ns
    nrm_ref[1] = nd
    xw = jnp.dot(x_ref[...], w1_ref[...], preferred_element_type=jnp.float32)
    p1 = xw * ns
    p1_ref[0] = p1[:, :DH2]
    p1_ref[1] = p1[:, DH2:]


def _tc2_body(agg_ref, nrm_ref, b1_ref, w2_ref, p2_ref):
    a = jnp.concatenate([agg_ref[0], agg_ref[1]], axis=-1)
    h = jnp.maximum(a * nrm_ref[1] + b1_ref[...], 0.0)
    hw = jnp.dot(h, w2_ref[...], preferred_element_type=jnp.float32)
    p2 = hw * nrm_ref[0]
    p2_ref[0] = p2[:, :DO2]
    p2_ref[1] = p2[:, DO2:]


def _tc3_body(agg_ref, nrm_ref, b2_ref, o_ref):
    a = jnp.concatenate([agg_ref[0], agg_ref[1]], axis=-1)
    o_ref[...] = a * nrm_ref[1] + b2_ref[...]


_GRID = (NPAD // BR,)

_tc1 = pl.pallas_call(
    _tc1_body,
    grid=_GRID,
    in_specs=[
        pl.BlockSpec((2, 2, BR, 1), lambda i: (0, 0, i, 0)),
        pl.BlockSpec((BR, DIN), lambda i: (i, 0)),
        pl.BlockSpec((DIN, DH), lambda i: (0, 0)),
    ],
    out_specs=[
        pl.BlockSpec((2, BR, DH2), lambda i: (0, i, 0)),
        pl.BlockSpec((2, BR, 1), lambda i: (0, i, 0)),
    ],
    out_shape=[
        jax.ShapeDtypeStruct((2, NPAD, DH2), jnp.float32),
        jax.ShapeDtypeStruct((2, NPAD, 1), jnp.float32),
    ],
)

_tc2 = pl.pallas_call(
    _tc2_body,
    grid=_GRID,
    in_specs=[
        pl.BlockSpec((2, BR, DH2), lambda i: (0, i, 0)),
        pl.BlockSpec((2, BR, 1), lambda i: (0, i, 0)),
        pl.BlockSpec((1, DH), lambda i: (0, 0)),
        pl.BlockSpec((DH, DOUT), lambda i: (0, 0)),
    ],
    out_specs=pl.BlockSpec((2, BR, DO2), lambda i: (0, i, 0)),
    out_shape=jax.ShapeDtypeStruct((2, NPAD, DO2), jnp.float32),
)

_tc3 = pl.pallas_call(
    _tc3_body,
    grid=_GRID,
    in_specs=[
        pl.BlockSpec((2, BR, DO2), lambda i: (0, i, 0)),
        pl.BlockSpec((2, BR, 1), lambda i: (0, i, 0)),
        pl.BlockSpec((1, DOUT), lambda i: (0, 0)),
    ],
    out_specs=pl.BlockSpec((BR, DOUT), lambda i: (i, 0)),
    out_shape=jax.ShapeDtypeStruct((NPAD, DOUT), jnp.float32),
)


def kernel(features, edge_index, W1, b1, W2, b2):
    pad = jnp.full((EPAD - EE,), NN, jnp.int32)
    srcp = jnp.concatenate([edge_index[0], pad])
    dstp = jnp.concatenate([edge_index[1], pad])
    srcp_deg = srcp.reshape(32, NCHD, CH)
    dstp_deg = dstp.reshape(32, NCHD, CH)
    srcp_t = srcp.reshape(16, NCHP, CH)
    dstp_t = dstp.reshape(16, NCHP, CH)
    x_pad = jnp.pad(features, ((0, NPAD - NN), (0, 0)))

    degp = _deg_call(srcp_deg, dstp_deg)
    degcol = degp.reshape(2, 2, NPAD, 1)
    p1, nrm = _tc1(degcol, x_pad, W1)
    agg1 = _prop_h(p1, srcp_t, dstp_t)
    p2 = _tc2(agg1, nrm, b1.reshape(1, DH), W2)
    agg2 = _prop_o(p2, srcp_t, dstp_t)
    outp = _tc3(agg2, nrm, b2.reshape(1, DOUT))
    return outp[:NN]


# speedup vs baseline: 9.3735x; 1.0049x over previous
import jax
import jax.numpy as jnp
from jax import lax
from jax.experimental import pallas as pl
from jax.experimental.pallas import tpu as pltpu
from jax.experimental.pallas import tpu_sc as plsc

NN = 10000
EE = 320000
DIN = 128
DH = 128
DOUT = 64

NPAD = 10240
CH = 128
EPAD = 327680
NCHD = 80
NCHP = 160
RPT = NPAD // 16

_MESH = dict(core_axis_name="c", subcore_axis_name="s")


def _deg_body(srcp3, dstp3, out, sidx, didx, onesv, buf, dsrc_sh, ddst_sh,
              sems, semd):
    c = lax.axis_index("c")
    s = lax.axis_index("s")
    wid = s * 2 + c

    def zbody(i, _):
        buf[pl.ds(i * 16, 16)] = jnp.zeros((16,), jnp.float32)
        return 0

    lax.fori_loop(0, RPT // 16, zbody, 0)
    for i in range(CH // 16):
        onesv[pl.ds(i * 16, 16)] = jnp.ones((16,), jnp.float32)
    pltpu.sync_copy(srcp3.at[wid], sidx)
    pltpu.sync_copy(dstp3.at[wid], didx)
    pltpu.sync_copy(buf, dsrc_sh.at[pl.ds(s * RPT, RPT)])
    pltpu.sync_copy(buf, ddst_sh.at[pl.ds(s * RPT, RPT)])
    plsc.subcore_barrier()

    def ebody(t, _):
        pltpu.async_copy(onesv, dsrc_sh.at[sidx.at[t]], sems, add=True)
        pltpu.async_copy(onesv, ddst_sh.at[didx.at[t]], semd, add=True)

        @pl.when(t > 0)
        def _():
            pltpu.make_async_copy(onesv, dsrc_sh.at[sidx.at[t - 1]], sems).wait()
            pltpu.make_async_copy(onesv, ddst_sh.at[didx.at[t - 1]], semd).wait()

        return 0

    lax.fori_loop(0, NCHD, ebody, 0)
    pltpu.make_async_copy(onesv, dsrc_sh.at[sidx.at[NCHD - 1]], sems).wait()
    pltpu.make_async_copy(onesv, ddst_sh.at[didx.at[NCHD - 1]], semd).wait()
    plsc.subcore_barrier()

    pltpu.sync_copy(dsrc_sh.at[pl.ds(s * RPT, RPT)], buf)
    pltpu.sync_copy(buf, out.at[c, 0, pl.ds(s * RPT, RPT)])
    pltpu.sync_copy(ddst_sh.at[pl.ds(s * RPT, RPT)], buf)
    pltpu.sync_copy(buf, out.at[c, 1, pl.ds(s * RPT, RPT)])


_deg_call = pl.kernel(
    _deg_body,
    out_type=jax.ShapeDtypeStruct((2, 2, NPAD), jnp.float32),
    mesh=plsc.VectorSubcoreMesh(**_MESH),
    scratch_types=[
        pltpu.VMEM((NCHD, CH), jnp.int32),
        pltpu.VMEM((NCHD, CH), jnp.int32),
        pltpu.VMEM((CH,), jnp.float32),
        pltpu.VMEM((RPT,), jnp.float32),
        pltpu.VMEM_SHARED((NPAD,), jnp.float32),
        pltpu.VMEM_SHARED((NPAD,), jnp.float32),
        pltpu.SemaphoreType.DMA,
        pltpu.SemaphoreType.DMA,
    ],
)


def _make_prop(D2):
    IBK = 16
    NBLK = NCHP // IBK

    def _prop_body(p_hbm, srcp3, dstp3, out, sidx, di0, di1, r0, r1,
                   tbl, agg, g0, g1, c0, c1, i0, i1):
        rows = (r0, r1)
        dbuf = (di0, di1)
        gsem = (g0, g1)
        csem = (c0, c1)
        isem = (i0, i1)
        c = lax.axis_index("c")
        s = lax.axis_index("s")

        def zrow(i, _):
            for k in range(D2 // 16):
                r0[i, pl.ds(k * 16, 16)] = jnp.zeros((16,), jnp.float32)
            return 0

        lax.fori_loop(0, CH, zrow, 0)
        for r in range(RPT // CH):
            pltpu.sync_copy(r0, agg.at[pl.ds(s * RPT + r * CH, CH)])
        for r in range(RPT // CH):
            off = s * RPT + r * CH
            pltpu.sync_copy(p_hbm.at[c, pl.ds(off, CH)], r0)
            pltpu.sync_copy(r0, tbl.at[pl.ds(off, CH)])
        pltpu.sync_copy(srcp3.at[s], sidx)
        pltpu.sync_copy(dstp3.at[s, pl.ds(0, IBK)], di0)
        plsc.subcore_barrier()

        pltpu.async_copy(tbl.at[sidx.at[0]], r0, g0)

        def _dblk(blk):
            return dstp3.at[s, pl.ds(blk * IBK, IBK)]

        def pair_body(u, _):
            for par in range(2):
                blk = 2 * u + par
                j0 = blk * IBK
                dref = dbuf[par]
                oref = dbuf[1 - par]
                if par == 0:
                    @pl.when(u > 0)
                    def _(oref=oref):
                        pltpu.make_async_copy(
                            rows[1], agg.at[oref.at[IBK - 1]], csem[1]).wait()
                else:
                    pltpu.make_async_copy(
                        rows[1], agg.at[oref.at[IBK - 1]], csem[1]).wait()

                @pl.when(blk + 1 < NBLK)
                def _(blk=blk, oref=oref, par=par):
                    pltpu.async_copy(_dblk(blk + 1), oref, isem[1 - par])

                if par == 0:
                    @pl.when(u > 0)
                    def _(dref=dref, par=par, blk=blk):
                        pltpu.make_async_copy(_dblk(blk), dref,
                                              isem[par]).wait()
                else:
                    pltpu.make_async_copy(_dblk(blk), dref, isem[par]).wait()

                for q in range(IBK):
                    j = j0 + q
                    b = q % 2
                    pltpu.make_async_copy(
                        tbl.at[sidx.at[j]], rows[b], gsem[b]).wait()
                    pltpu.async_copy(rows[b], agg.at[dref.at[q]], csem[b],
                                     add=True)
                    if q >= 1:
                        pltpu.make_async_copy(
                            rows[1 - b], agg.at[dref.at[q - 1]],
                            csem[1 - b]).wait()

                    @pl.when(j + 1 < NCHP)
                    def _(j=j, b=b):
                        pltpu.async_copy(
                            tbl.at[sidx.at[j + 1]], rows[1 - b], gsem[1 - b])
            return 0

        lax.fori_loop(0, NBLK // 2, pair_body, 0)
        pltpu.make_async_copy(
            rows[1], agg.at[di1.at[IBK - 1]], csem[1]).wait()
        plsc.subcore_barrier()

        for r in range(RPT // CH):
            off = s * RPT + r * CH
            pltpu.sync_copy(agg.at[pl.ds(off, CH)], r0)
            pltpu.sync_copy(r0, out.at[c, pl.ds(off, CH)])

    return pl.kernel(
        _prop_body,
        out_type=jax.ShapeDtypeStruct((2, NPAD, D2), jnp.float32),
        mesh=plsc.VectorSubcoreMesh(**_MESH),
        compiler_params=pltpu.CompilerParams(use_tc_tiling_on_sc=False),
        scratch_types=(
            [pltpu.VMEM((NCHP, CH), jnp.int32)]
            + [pltpu.VMEM((IBK, CH), jnp.int32)] * 2
            + [pltpu.VMEM((CH, D2), jnp.float32)] * 2
            + [pltpu.VMEM_SHARED((NPAD, D2), jnp.float32)] * 2
            + [pltpu.SemaphoreType.DMA] * 6
        ),
    )


_prop_h = _make_prop(DH // 2)
_prop_o = _make_prop(DOUT // 2)


BR = 512
DH2 = DH // 2
DO2 = DOUT // 2


def _tcz_body(x_ref, w1_ref, z_ref):
    z_ref[...] = jnp.dot(x_ref[...], w1_ref[...],
                         preferred_element_type=jnp.float32)


def _tc1_body(deg_ref, z_ref, p1_ref, nrm_ref):
    d = deg_ref[...]
    dsrc = d[0, 0] + d[1, 0]
    ddst = d[0, 1] + d[1, 1]
    ns = jnp.where(dsrc > 0, lax.rsqrt(jnp.maximum(dsrc, 1.0)), 0.0)
    nd = jnp.where(ddst > 0, lax.rsqrt(jnp.maximum(ddst, 1.0)), 0.0)
    nrm_ref[0] = ns
    nrm_ref[1] = nd
    p1 = z_ref[...] * ns
    p1_ref[0] = p1[:, :DH2]
    p1_ref[1] = p1[:, DH2:]


def _tc2_body(agg_ref, nrm_ref, b1_ref, w2_ref, p2_ref):
    a = jnp.concatenate([agg_ref[0], agg_ref[1]], axis=-1)
    h = jnp.maximum(a * nrm_ref[1] + b1_ref[...], 0.0)
    hw = jnp.dot(h, w2_ref[...], preferred_element_type=jnp.float32)
    p2 = hw * nrm_ref[0]
    p2_ref[0] = p2[:, :DO2]
    p2_ref[1] = p2[:, DO2:]


def _tc3_body(agg_ref, nrm_ref, b2_ref, o_ref):
    a = jnp.concatenate([agg_ref[0], agg_ref[1]], axis=-1)
    o_ref[...] = a * nrm_ref[1] + b2_ref[...]


_GRID = (NPAD // BR,)

_tcz = pl.pallas_call(
    _tcz_body,
    grid=_GRID,
    in_specs=[
        pl.BlockSpec((BR, DIN), lambda i: (i, 0)),
        pl.BlockSpec((DIN, DH), lambda i: (0, 0)),
    ],
    out_specs=pl.BlockSpec((BR, DH), lambda i: (i, 0)),
    out_shape=jax.ShapeDtypeStruct((NPAD, DH), jnp.float32),
)

_tc1 = pl.pallas_call(
    _tc1_body,
    grid=_GRID,
    in_specs=[
        pl.BlockSpec((2, 2, BR, 1), lambda i: (0, 0, i, 0)),
        pl.BlockSpec((BR, DH), lambda i: (i, 0)),
    ],
    out_specs=[
        pl.BlockSpec((2, BR, DH2), lambda i: (0, i, 0)),
        pl.BlockSpec((2, BR, 1), lambda i: (0, i, 0)),
    ],
    out_shape=[
        jax.ShapeDtypeStruct((2, NPAD, DH2), jnp.float32),
        jax.ShapeDtypeStruct((2, NPAD, 1), jnp.float32),
    ],
)

_tc2 = pl.pallas_call(
    _tc2_body,
    grid=_GRID,
    in_specs=[
        pl.BlockSpec((2, BR, DH2), lambda i: (0, i, 0)),
        pl.BlockSpec((2, BR, 1), lambda i: (0, i, 0)),
        pl.BlockSpec((1, DH), lambda i: (0, 0)),
        pl.BlockSpec((DH, DOUT), lambda i: (0, 0)),
    ],
    out_specs=pl.BlockSpec((2, BR, DO2), lambda i: (0, i, 0)),
    out_shape=jax.ShapeDtypeStruct((2, NPAD, DO2), jnp.float32),
)

_tc3 = pl.pallas_call(
    _tc3_body,
    grid=_GRID,
    in_specs=[
        pl.BlockSpec((2, BR, DO2), lambda i: (0, i, 0)),
        pl.BlockSpec((2, BR, 1), lambda i: (0, i, 0)),
        pl.BlockSpec((1, DOUT), lambda i: (0, 0)),
    ],
    out_specs=pl.BlockSpec((BR, DOUT), lambda i: (i, 0)),
    out_shape=jax.ShapeDtypeStruct((NPAD, DOUT), jnp.float32),
)


def kernel(features, edge_index, W1, b1, W2, b2):
    pad = jnp.full((EPAD - EE,), NN, jnp.int32)
    srcp = jnp.concatenate([edge_index[0], pad])
    dstp = jnp.concatenate([edge_index[1], pad])
    srcp_deg = srcp.reshape(32, NCHD, CH)
    dstp_deg = dstp.reshape(32, NCHD, CH)
    srcp_t = srcp.reshape(16, NCHP, CH)
    dstp_t = dstp.reshape(16, NCHP, CH)
    x_pad = jnp.pad(features, ((0, NPAD - NN), (0, 0)))

    z1 = _tcz(x_pad, W1)
    degp = _deg_call(srcp_deg, dstp_deg)
    degcol = degp.reshape(2, 2, NPAD, 1)
    p1, nrm = _tc1(degcol, z1)
    agg1 = _prop_h(p1, srcp_t, dstp_t)
    p2 = _tc2(agg1, nrm, b1.reshape(1, DH), W2)
    agg2 = _prop_o(p2, srcp_t, dstp_t)
    outp = _tc3(agg2, nrm, b2.reshape(1, DOUT))
    return outp[:NN]


# speedup vs baseline: 9.3874x; 1.0015x over previous
import jax
import jax.numpy as jnp
from jax import lax
from jax.experimental import pallas as pl
from jax.experimental.pallas import tpu as pltpu
from jax.experimental.pallas import tpu_sc as plsc

NN = 10000
EE = 320000
DIN = 128
DH = 128
DOUT = 64

NPAD = 10240
CH = 128
EPAD = 327680
NCHD = 80
NCHP = 160
RPT = NPAD // 16

_MESH = dict(core_axis_name="c", subcore_axis_name="s")


def _deg_body(srcp3, dstp3, out, sidx, didx, onesv, buf, dsrc_sh, ddst_sh,
              sems, semd):
    c = lax.axis_index("c")
    s = lax.axis_index("s")
    wid = s * 2 + c

    def zbody(i, _):
        buf[pl.ds(i * 16, 16)] = jnp.zeros((16,), jnp.float32)
        return 0

    lax.fori_loop(0, RPT // 16, zbody, 0)
    for i in range(CH // 16):
        onesv[pl.ds(i * 16, 16)] = jnp.ones((16,), jnp.float32)
    pltpu.sync_copy(srcp3.at[wid], sidx)
    pltpu.sync_copy(dstp3.at[wid], didx)
    pltpu.sync_copy(buf, dsrc_sh.at[pl.ds(s * RPT, RPT)])
    pltpu.sync_copy(buf, ddst_sh.at[pl.ds(s * RPT, RPT)])
    plsc.subcore_barrier()

    def ebody(t, _):
        pltpu.async_copy(onesv, dsrc_sh.at[sidx.at[t]], sems, add=True)
        pltpu.async_copy(onesv, ddst_sh.at[didx.at[t]], semd, add=True)

        @pl.when(t > 2)
        def _():
            pltpu.make_async_copy(onesv, dsrc_sh.at[sidx.at[t - 3]], sems).wait()
            pltpu.make_async_copy(onesv, ddst_sh.at[didx.at[t - 3]], semd).wait()

        return 0

    lax.fori_loop(0, NCHD, ebody, 0)
    for k in range(NCHD - 3, NCHD):
        pltpu.make_async_copy(onesv, dsrc_sh.at[sidx.at[k]], sems).wait()
        pltpu.make_async_copy(onesv, ddst_sh.at[didx.at[k]], semd).wait()
    plsc.subcore_barrier()

    pltpu.sync_copy(dsrc_sh.at[pl.ds(s * RPT, RPT)], buf)
    pltpu.sync_copy(buf, out.at[c, 0, pl.ds(s * RPT, RPT)])
    pltpu.sync_copy(ddst_sh.at[pl.ds(s * RPT, RPT)], buf)
    pltpu.sync_copy(buf, out.at[c, 1, pl.ds(s * RPT, RPT)])


_deg_call = pl.kernel(
    _deg_body,
    out_type=jax.ShapeDtypeStruct((2, 2, NPAD), jnp.float32),
    mesh=plsc.VectorSubcoreMesh(**_MESH),
    scratch_types=[
        pltpu.VMEM((NCHD, CH), jnp.int32),
        pltpu.VMEM((NCHD, CH), jnp.int32),
        pltpu.VMEM((CH,), jnp.float32),
        pltpu.VMEM((RPT,), jnp.float32),
        pltpu.VMEM_SHARED((NPAD,), jnp.float32),
        pltpu.VMEM_SHARED((NPAD,), jnp.float32),
        pltpu.SemaphoreType.DMA,
        pltpu.SemaphoreType.DMA,
    ],
)


def _make_prop(D2):
    IBK = 16
    NBLK = NCHP // IBK

    def _prop_body(p_hbm, srcp3, dstp3, out, sidx, di0, di1, r0, r1,
                   tbl, agg, g0, g1, c0, c1, i0, i1):
        rows = (r0, r1)
        dbuf = (di0, di1)
        gsem = (g0, g1)
        csem = (c0, c1)
        isem = (i0, i1)
        c = lax.axis_index("c")
        s = lax.axis_index("s")

        def zrow(i, _):
            for k in range(D2 // 16):
                r0[i, pl.ds(k * 16, 16)] = jnp.zeros((16,), jnp.float32)
            return 0

        lax.fori_loop(0, CH, zrow, 0)
        for r in range(RPT // CH):
            pltpu.sync_copy(r0, agg.at[pl.ds(s * RPT + r * CH, CH)])
        for r in range(RPT // CH):
            off = s * RPT + r * CH
            pltpu.sync_copy(p_hbm.at[c, pl.ds(off, CH)], r0)
            pltpu.sync_copy(r0, tbl.at[pl.ds(off, CH)])
        pltpu.sync_copy(srcp3.at[s], sidx)
        pltpu.sync_copy(dstp3.at[s, pl.ds(0, IBK)], di0)
        plsc.subcore_barrier()

        pltpu.async_copy(tbl.at[sidx.at[0]], r0, g0)

        def _dblk(blk):
            return dstp3.at[s, pl.ds(blk * IBK, IBK)]

        def pair_body(u, _):
            for par in range(2):
                blk = 2 * u + par
                j0 = blk * IBK
                dref = dbuf[par]
                oref = dbuf[1 - par]
                if par == 0:
                    @pl.when(u > 0)
                    def _(oref=oref):
                        pltpu.make_async_copy(
                            rows[1], agg.at[oref.at[IBK - 1]], csem[1]).wait()
                else:
                    pltpu.make_async_copy(
                        rows[1], agg.at[oref.at[IBK - 1]], csem[1]).wait()

                @pl.when(blk + 1 < NBLK)
                def _(blk=blk, oref=oref, par=par):
                    pltpu.async_copy(_dblk(blk + 1), oref, isem[1 - par])

                if par == 0:
                    @pl.when(u > 0)
                    def _(dref=dref, par=par, blk=blk):
                        pltpu.make_async_copy(_dblk(blk), dref,
                                              isem[par]).wait()
                else:
                    pltpu.make_async_copy(_dblk(blk), dref, isem[par]).wait()

                for q in range(IBK):
                    j = j0 + q
                    b = q % 2
                    pltpu.make_async_copy(
                        tbl.at[sidx.at[j]], rows[b], gsem[b]).wait()
                    pltpu.async_copy(rows[b], agg.at[dref.at[q]], csem[b],
                                     add=True)
                    if q >= 1:
                        pltpu.make_async_copy(
                            rows[1 - b], agg.at[dref.at[q - 1]],
                            csem[1 - b]).wait()

                    @pl.when(j + 1 < NCHP)
                    def _(j=j, b=b):
                        pltpu.async_copy(
                            tbl.at[sidx.at[j + 1]], rows[1 - b], gsem[1 - b])
            return 0

        lax.fori_loop(0, NBLK // 2, pair_body, 0)
        pltpu.make_async_copy(
            rows[1], agg.at[di1.at[IBK - 1]], csem[1]).wait()
        plsc.subcore_barrier()

        for r in range(RPT // CH):
            off = s * RPT + r * CH
            pltpu.sync_copy(agg.at[pl.ds(off, CH)], r0)
            pltpu.sync_copy(r0, out.at[c, pl.ds(off, CH)])

    return pl.kernel(
        _prop_body,
        out_type=jax.ShapeDtypeStruct((2, NPAD, D2), jnp.float32),
        mesh=plsc.VectorSubcoreMesh(**_MESH),
        compiler_params=pltpu.CompilerParams(use_tc_tiling_on_sc=False),
        scratch_types=(
            [pltpu.VMEM((NCHP, CH), jnp.int32)]
            + [pltpu.VMEM((IBK, CH), jnp.int32)] * 2
            + [pltpu.VMEM((CH, D2), jnp.float32)] * 2
            + [pltpu.VMEM_SHARED((NPAD, D2), jnp.float32)] * 2
            + [pltpu.SemaphoreType.DMA] * 6
        ),
    )


_prop_h = _make_prop(DH // 2)
_prop_o = _make_prop(DOUT // 2)


BR = 512
DH2 = DH // 2
DO2 = DOUT // 2


def _tcz_body(x_ref, w1_ref, z_ref):
    z_ref[...] = jnp.dot(x_ref[...], w1_ref[...],
                         preferred_element_type=jnp.float32)


def _tc1_body(deg_ref, z_ref, p1_ref, nrm_ref):
    d = deg_ref[...]
    dsrc = d[0, 0] + d[1, 0]
    ddst = d[0, 1] + d[1, 1]
    ns = jnp.where(dsrc > 0, lax.rsqrt(jnp.maximum(dsrc, 1.0)), 0.0)
    nd = jnp.where(ddst > 0, lax.rsqrt(jnp.maximum(ddst, 1.0)), 0.0)
    nrm_ref[0] = ns
    nrm_ref[1] = nd
    p1 = z_ref[...] * ns
    p1_ref[0] = p1[:, :DH2]
    p1_ref[1] = p1[:, DH2:]


def _tc2_body(agg_ref, nrm_ref, b1_ref, w2_ref, p2_ref):
    a = jnp.concatenate([agg_ref[0], agg_ref[1]], axis=-1)
    h = jnp.maximum(a * nrm_ref[1] + b1_ref[...], 0.0)
    hw = jnp.dot(h, w2_ref[...], preferred_element_type=jnp.float32)
    p2 = hw * nrm_ref[0]
    p2_ref[0] = p2[:, :DO2]
    p2_ref[1] = p2[:, DO2:]


def _tc3_body(agg_ref, nrm_ref, b2_ref, o_ref):
    a = jnp.concatenate([agg_ref[0], agg_ref[1]], axis=-1)
    o_ref[...] = a * nrm_ref[1] + b2_ref[...]


_GRID = (NPAD // BR,)

_tcz = pl.pallas_call(
    _tcz_body,
    grid=_GRID,
    in_specs=[
        pl.BlockSpec((BR, DIN), lambda i: (i, 0)),
        pl.BlockSpec((DIN, DH), lambda i: (0, 0)),
    ],
    out_specs=pl.BlockSpec((BR, DH), lambda i: (i, 0)),
    out_shape=jax.ShapeDtypeStruct((NPAD, DH), jnp.float32),
)

_tc1 = pl.pallas_call(
    _tc1_body,
    grid=_GRID,
    in_specs=[
        pl.BlockSpec((2, 2, BR, 1), lambda i: (0, 0, i, 0)),
        pl.BlockSpec((BR, DH), lambda i: (i, 0)),
    ],
    out_specs=[
        pl.BlockSpec((2, BR, DH2), lambda i: (0, i, 0)),
        pl.BlockSpec((2, BR, 1), lambda i: (0, i, 0)),
    ],
    out_shape=[
        jax.ShapeDtypeStruct((2, NPAD, DH2), jnp.float32),
        jax.ShapeDtypeStruct((2, NPAD, 1), jnp.float32),
    ],
)

_tc2 = pl.pallas_call(
    _tc2_body,
    grid=_GRID,
    in_specs=[
        pl.BlockSpec((2, BR, DH2), lambda i: (0, i, 0)),
        pl.BlockSpec((2, BR, 1), lambda i: (0, i, 0)),
        pl.BlockSpec((1, DH), lambda i: (0, 0)),
        pl.BlockSpec((DH, DOUT), lambda i: (0, 0)),
    ],
    out_specs=pl.BlockSpec((2, BR, DO2), lambda i: (0, i, 0)),
    out_shape=jax.ShapeDtypeStruct((2, NPAD, DO2), jnp.float32),
)

_tc3 = pl.pallas_call(
    _tc3_body,
    grid=_GRID,
    in_specs=[
        pl.BlockSpec((2, BR, DO2), lambda i: (0, i, 0)),
        pl.BlockSpec((2, BR, 1), lambda i: (0, i, 0)),
        pl.BlockSpec((1, DOUT), lambda i: (0, 0)),
    ],
    out_specs=pl.BlockSpec((BR, DOUT), lambda i: (i, 0)),
    out_shape=jax.ShapeDtypeStruct((NPAD, DOUT), jnp.float32),
)


def kernel(features, edge_index, W1, b1, W2, b2):
    pad = jnp.full((EPAD - EE,), NN, jnp.int32)
    srcp = jnp.concatenate([edge_index[0], pad])
    dstp = jnp.concatenate([edge_index[1], pad])
    srcp_deg = srcp.reshape(32, NCHD, CH)
    dstp_deg = dstp.reshape(32, NCHD, CH)
    srcp_t = srcp.reshape(16, NCHP, CH)
    dstp_t = dstp.reshape(16, NCHP, CH)
    x_pad = jnp.pad(features, ((0, NPAD - NN), (0, 0)))

    z1 = _tcz(x_pad, W1)
    degp = _deg_call(srcp_deg, dstp_deg)
    degcol = degp.reshape(2, 2, NPAD, 1)
    p1, nrm = _tc1(degcol, z1)
    agg1 = _prop_h(p1, srcp_t, dstp_t)
    p2 = _tc2(agg1, nrm, b1.reshape(1, DH), W2)
    agg2 = _prop_o(p2, srcp_t, dstp_t)
    outp = _tc3(agg2, nrm, b2.reshape(1, DOUT))
    return outp[:NN]


# speedup vs baseline: 9.4886x; 1.0108x over previous
import jax
import jax.numpy as jnp
from jax import lax
from jax.experimental import pallas as pl
from jax.experimental.pallas import tpu as pltpu
from jax.experimental.pallas import tpu_sc as plsc

NN = 10000
EE = 320000
DIN = 128
DH = 128
DOUT = 64

NPAD = 10240
CH = 128
EPAD = 327680
NCHD = 80
NCHP = 160
RPT = NPAD // 16

_MESH = dict(core_axis_name="c", subcore_axis_name="s")


def _deg_body(srcp3, dstp3, out, sidx, didx, onesv, buf, dsrc_sh, ddst_sh,
              sems, semd):
    c = lax.axis_index("c")
    s = lax.axis_index("s")
    wid = s * 2 + c

    def zbody(i, _):
        buf[pl.ds(i * 16, 16)] = jnp.zeros((16,), jnp.float32)
        return 0

    lax.fori_loop(0, RPT // 16, zbody, 0)
    for i in range(CH // 16):
        onesv[pl.ds(i * 16, 16)] = jnp.ones((16,), jnp.float32)
    pltpu.sync_copy(srcp3.at[wid], sidx)
    pltpu.sync_copy(dstp3.at[wid], didx)
    pltpu.sync_copy(buf, dsrc_sh.at[pl.ds(s * RPT, RPT)])
    pltpu.sync_copy(buf, ddst_sh.at[pl.ds(s * RPT, RPT)])
    plsc.subcore_barrier()

    def ebody(t, _):
        pltpu.async_copy(onesv, dsrc_sh.at[sidx.at[t]], sems, add=True)
        pltpu.async_copy(onesv, ddst_sh.at[didx.at[t]], semd, add=True)

        @pl.when(t > 2)
        def _():
            pltpu.make_async_copy(onesv, dsrc_sh.at[sidx.at[t - 3]], sems).wait()
            pltpu.make_async_copy(onesv, ddst_sh.at[didx.at[t - 3]], semd).wait()

        return 0

    lax.fori_loop(0, NCHD, ebody, 0)
    for k in range(NCHD - 3, NCHD):
        pltpu.make_async_copy(onesv, dsrc_sh.at[sidx.at[k]], sems).wait()
        pltpu.make_async_copy(onesv, ddst_sh.at[didx.at[k]], semd).wait()
    plsc.subcore_barrier()

    pltpu.sync_copy(dsrc_sh.at[pl.ds(s * RPT, RPT)], buf)
    pltpu.sync_copy(buf, out.at[c, 0, pl.ds(s * RPT, RPT)])
    pltpu.sync_copy(ddst_sh.at[pl.ds(s * RPT, RPT)], buf)
    pltpu.sync_copy(buf, out.at[c, 1, pl.ds(s * RPT, RPT)])


_deg_call = pl.kernel(
    _deg_body,
    out_type=jax.ShapeDtypeStruct((2, 2, NPAD), jnp.float32),
    mesh=plsc.VectorSubcoreMesh(**_MESH),
    scratch_types=[
        pltpu.VMEM((NCHD, CH), jnp.int32),
        pltpu.VMEM((NCHD, CH), jnp.int32),
        pltpu.VMEM((CH,), jnp.float32),
        pltpu.VMEM((RPT,), jnp.float32),
        pltpu.VMEM_SHARED((NPAD,), jnp.float32),
        pltpu.VMEM_SHARED((NPAD,), jnp.float32),
        pltpu.SemaphoreType.DMA,
        pltpu.SemaphoreType.DMA,
    ],
)


def _rsqrt16(m):
    ib = plsc.bitcast(m, jnp.int32)
    y = plsc.bitcast(
        jnp.full((16,), 0x5F3759DF, jnp.int32) - (ib >> 1), jnp.float32)
    for _ in range(3):
        y = y * (1.5 - 0.5 * m * y * y)
    return y


def _make_prop(D2, fold_out=False):
    IBK = 16
    NBLK = NCHP // IBK

    def _prop_body(p_hbm, srcp3, dstp3, *args):
        if fold_out:
            (degp, b2h, out, sidx, di0, di1, r0, r1, tbl, agg,
             g0, g1, c0, c1, i0, i1, d0, d1, ndbuf, b2buf) = args
        else:
            (out, sidx, di0, di1, r0, r1, tbl, agg,
             g0, g1, c0, c1, i0, i1) = args
        rows = (r0, r1)
        dbuf = (di0, di1)
        gsem = (g0, g1)
        csem = (c0, c1)
        isem = (i0, i1)
        c = lax.axis_index("c")
        s = lax.axis_index("s")

        def zrow(i, _):
            for k in range(D2 // 16):
                r0[i, pl.ds(k * 16, 16)] = jnp.zeros((16,), jnp.float32)
            return 0

        lax.fori_loop(0, CH, zrow, 0)
        for r in range(RPT // CH):
            pltpu.sync_copy(r0, agg.at[pl.ds(s * RPT + r * CH, CH)])
        for r in range(RPT // CH):
            off = s * RPT + r * CH
            pltpu.sync_copy(p_hbm.at[c, pl.ds(off, CH)], r0)
            pltpu.sync_copy(r0, tbl.at[pl.ds(off, CH)])
        pltpu.sync_copy(srcp3.at[s], sidx)
        pltpu.sync_copy(dstp3.at[s, pl.ds(0, IBK)], di0)
        if fold_out:
            pltpu.sync_copy(degp.at[0, 1, pl.ds(s * RPT, RPT)], d0)
            pltpu.sync_copy(degp.at[1, 1, pl.ds(s * RPT, RPT)], d1)
            pltpu.sync_copy(b2h.at[c], b2buf)

            def ndbody(v, _):
                x = d0[pl.ds(16 * v, 16)] + d1[pl.ds(16 * v, 16)]
                y = _rsqrt16(jnp.maximum(x, 1.0))
                ndbuf[pl.ds(16 * v, 16)] = jnp.where(x > 0, y, 0.0)
                return 0

            lax.fori_loop(0, RPT // 16, ndbody, 0)
        plsc.subcore_barrier()

        pltpu.async_copy(tbl.at[sidx.at[0]], r0, g0)

        def _dblk(blk):
            return dstp3.at[s, pl.ds(blk * IBK, IBK)]

        def pair_body(u, _):
            for par in range(2):
                blk = 2 * u + par
                j0 = blk * IBK
                dref = dbuf[par]
                oref = dbuf[1 - par]
                if par == 0:
                    @pl.when(u > 0)
                    def _(oref=oref):
                        pltpu.make_async_copy(
                            rows[1], agg.at[oref.at[IBK - 1]], csem[1]).wait()
                else:
                    pltpu.make_async_copy(
                        rows[1], agg.at[oref.at[IBK - 1]], csem[1]).wait()

                @pl.when(blk + 1 < NBLK)
                def _(blk=blk, oref=oref, par=par):
                    pltpu.async_copy(_dblk(blk + 1), oref, isem[1 - par])

                if par == 0:
                    @pl.when(u > 0)
                    def _(dref=dref, par=par, blk=blk):
                        pltpu.make_async_copy(_dblk(blk), dref,
                                              isem[par]).wait()
                else:
                    pltpu.make_async_copy(_dblk(blk), dref, isem[par]).wait()

                for q in range(IBK):
                    j = j0 + q
                    b = q % 2
                    pltpu.make_async_copy(
                        tbl.at[sidx.at[j]], rows[b], gsem[b]).wait()
                    pltpu.async_copy(rows[b], agg.at[dref.at[q]], csem[b],
                                     add=True)
                    if q >= 1:
                        pltpu.make_async_copy(
                            rows[1 - b], agg.at[dref.at[q - 1]],
                            csem[1 - b]).wait()

                    @pl.when(j + 1 < NCHP)
                    def _(j=j, b=b):
                        pltpu.async_copy(
                            tbl.at[sidx.at[j + 1]], rows[1 - b], gsem[1 - b])
            return 0

        lax.fori_loop(0, NBLK // 2, pair_body, 0)
        pltpu.make_async_copy(
            rows[1], agg.at[di1.at[IBK - 1]], csem[1]).wait()
        plsc.subcore_barrier()

        for r in range(RPT // CH):
            off = s * RPT + r * CH
            pltpu.sync_copy(agg.at[pl.ds(off, CH)], r0)
            if fold_out:
                def rowbody(row, _, r=r):
                    ndv = plsc.load_gather(
                        ndbuf, [jnp.full((16,), r * CH + row, jnp.int32)])
                    for k in range(D2 // 16):
                        a = (r0[row, pl.ds(16 * k, 16)] * ndv
                             + b2buf[pl.ds(16 * k, 16)])
                        r0[row, pl.ds(16 * k, 16)] = a
                    return 0

                lax.fori_loop(0, CH, rowbody, 0)
            pltpu.sync_copy(r0, out.at[c, pl.ds(off, CH)])

    return pl.kernel(
        _prop_body,
        out_type=jax.ShapeDtypeStruct((2, NPAD, D2), jnp.float32),
        mesh=plsc.VectorSubcoreMesh(**_MESH),
        compiler_params=(
            pltpu.CompilerParams(use_tc_tiling_on_sc=False,
                                 needs_layout_passes=False)
            if fold_out else
            pltpu.CompilerParams(use_tc_tiling_on_sc=False)),
        scratch_types=(
            [pltpu.VMEM((NCHP, CH), jnp.int32)]
            + [pltpu.VMEM((IBK, CH), jnp.int32)] * 2
            + [pltpu.VMEM((CH, D2), jnp.float32)] * 2
            + [pltpu.VMEM_SHARED((NPAD, D2), jnp.float32)] * 2
            + [pltpu.SemaphoreType.DMA] * 6
            + ([pltpu.VMEM((RPT,), jnp.float32)] * 3
               + [pltpu.VMEM((D2,), jnp.float32)] if fold_out else [])
        ),
    )


_prop_h = _make_prop(DH // 2)
_prop_o = _make_prop(DOUT // 2, fold_out=True)


BR = 512
DH2 = DH // 2
DO2 = DOUT // 2


def _tcz_body(x_ref, w1_ref, z_ref):
    z_ref[...] = jnp.dot(x_ref[...], w1_ref[...],
                         preferred_element_type=jnp.float32)


def _tc1_body(deg_ref, z_ref, p1_ref, nrm_ref):
    d = deg_ref[...]
    dsrc = d[0, 0] + d[1, 0]
    ddst = d[0, 1] + d[1, 1]
    ns = jnp.where(dsrc > 0, lax.rsqrt(jnp.maximum(dsrc, 1.0)), 0.0)
    nd = jnp.where(ddst > 0, lax.rsqrt(jnp.maximum(ddst, 1.0)), 0.0)
    nrm_ref[0] = ns
    nrm_ref[1] = nd
    p1 = z_ref[...] * ns
    p1_ref[0] = p1[:, :DH2]
    p1_ref[1] = p1[:, DH2:]


def _tc2_body(agg_ref, nrm_ref, b1_ref, w2_ref, p2_ref):
    a = jnp.concatenate([agg_ref[0], agg_ref[1]], axis=-1)
    h = jnp.maximum(a * nrm_ref[1] + b1_ref[...], 0.0)
    hw = jnp.dot(h, w2_ref[...], preferred_element_type=jnp.float32)
    p2 = hw * nrm_ref[0]
    p2_ref[0] = p2[:, :DO2]
    p2_ref[1] = p2[:, DO2:]


def _tc3_body(agg_ref, nrm_ref, b2_ref, o_ref):
    a = jnp.concatenate([agg_ref[0], agg_ref[1]], axis=-1)
    o_ref[...] = a * nrm_ref[1] + b2_ref[...]


_GRID = (NPAD // BR,)

_tcz = pl.pallas_call(
    _tcz_body,
    grid=_GRID,
    in_specs=[
        pl.BlockSpec((BR, DIN), lambda i: (i, 0)),
        pl.BlockSpec((DIN, DH), lambda i: (0, 0)),
    ],
    out_specs=pl.BlockSpec((BR, DH), lambda i: (i, 0)),
    out_shape=jax.ShapeDtypeStruct((NPAD, DH), jnp.float32),
)

_tc1 = pl.pallas_call(
    _tc1_body,
    grid=_GRID,
    in_specs=[
        pl.BlockSpec((2, 2, BR, 1), lambda i: (0, 0, i, 0)),
        pl.BlockSpec((BR, DH), lambda i: (i, 0)),
    ],
    out_specs=[
        pl.BlockSpec((2, BR, DH2), lambda i: (0, i, 0)),
        pl.BlockSpec((2, BR, 1), lambda i: (0, i, 0)),
    ],
    out_shape=[
        jax.ShapeDtypeStruct((2, NPAD, DH2), jnp.float32),
        jax.ShapeDtypeStruct((2, NPAD, 1), jnp.float32),
    ],
)

_tc2 = pl.pallas_call(
    _tc2_body,
    grid=_GRID,
    in_specs=[
        pl.BlockSpec((2, BR, DH2), lambda i: (0, i, 0)),
        pl.BlockSpec((2, BR, 1), lambda i: (0, i, 0)),
        pl.BlockSpec((1, DH), lambda i: (0, 0)),
        pl.BlockSpec((DH, DOUT), lambda i: (0, 0)),
    ],
    out_specs=pl.BlockSpec((2, BR, DO2), lambda i: (0, i, 0)),
    out_shape=jax.ShapeDtypeStruct((2, NPAD, DO2), jnp.float32),
)

_tc3 = pl.pallas_call(
    _tc3_body,
    grid=_GRID,
    in_specs=[
        pl.BlockSpec((2, BR, DO2), lambda i: (0, i, 0)),
        pl.BlockSpec((2, BR, 1), lambda i: (0, i, 0)),
        pl.BlockSpec((1, DOUT), lambda i: (0, 0)),
    ],
    out_specs=pl.BlockSpec((BR, DOUT), lambda i: (i, 0)),
    out_shape=jax.ShapeDtypeStruct((NPAD, DOUT), jnp.float32),
)


def kernel(features, edge_index, W1, b1, W2, b2):
    pad = jnp.full((EPAD - EE,), NN, jnp.int32)
    srcp = jnp.concatenate([edge_index[0], pad])
    dstp = jnp.concatenate([edge_index[1], pad])
    srcp_deg = srcp.reshape(32, NCHD, CH)
    dstp_deg = dstp.reshape(32, NCHD, CH)
    srcp_t = srcp.reshape(16, NCHP, CH)
    dstp_t = dstp.reshape(16, NCHP, CH)
    x_pad = jnp.pad(features, ((0, NPAD - NN), (0, 0)))

    z1 = _tcz(x_pad, W1)
    degp = _deg_call(srcp_deg, dstp_deg)
    degcol = degp.reshape(2, 2, NPAD, 1)
    p1, nrm = _tc1(degcol, z1)
    agg1 = _prop_h(p1, srcp_t, dstp_t)
    p2 = _tc2(agg1, nrm, b1.reshape(1, DH), W2)
    out2 = _prop_o(p2, srcp_t, dstp_t, degp, b2.reshape(2, DO2))
    return jnp.concatenate([out2[0], out2[1]], axis=1)[:NN]
